# baseline (speedup 1.0000x reference)
import functools

import jax
import jax.numpy as jnp
from jax import lax
from jax.experimental import pallas as pl
from jax.experimental.pallas import tpu as pltpu
from jax.experimental.pallas import tpu_sc as plsc

_NB = 16384
_NT = 50
_NF = 64
_V = 1000000
_QPW = 200
_H0 = 96
_H1 = 104


def _body(idx_flat, w2, ot, idx_raw, idx_g, rows_v, b_out,
          isem, gs0, gs1, os0, os1):
  cid = lax.axis_index("c")
  sid = lax.axis_index("s")
  w = sid * 2 + cid
  gses = (gs0, gs1)
  oses = (os0, os1)
  iot = lax.iota(jnp.int32, 16)
  qbase = w * _QPW

  def b_write(q, sl):
    t = q // 128
    bq = q % 128
    boff = pl.multiple_of(bq * 128, 128)
    return pltpu.make_async_copy(
        b_out.at[sl], ot.at[t, pl.ds(0, _NF), pl.ds(boff, 128)], oses[sl])

  for hh in range(2):
    nch = _H0 if hh == 0 else _H1
    hoff = 0 if hh == 0 else _H0 * 128

    idescs = [
        pltpu.make_async_copy(
            idx_flat.at[pl.ds(
                pl.multiple_of(w * _QPW * 128 + hoff + k * 1024, 1024),
                1024)],
            idx_raw.at[pl.ds(k * 1024, 1024)], isem)
        for k in range(nch * 128 // 1024)
    ]
    for d in idescs:
      d.start()
    for d in idescs:
      d.wait()

    @pl.loop(0, nch * 8)
    def _shift(k):
      v = idx_raw[pl.ds(k * 16, 16)]
      idx_g[pl.ds(k * 16, 16)] = lax.shift_right_logical(v, 1)

    def b_gather(lch, sl):
      ioff = pl.multiple_of(lch * 128, 128)
      return pltpu.make_async_copy(
          w2.at[idx_g.at[pl.ds(ioff, 128)]], rows_v.at[sl], gses[sl])

    @pl.loop(0, nch, step=2)
    def _bloop(ch0):
      for sl in range(2):
        lch = ch0 + sl

        @pl.when(ch0 >= 2)
        def _():
          b_write(qbase + hoff // 128 + lch - 2, sl).wait()

        b_gather(lch, sl).start()

      for sl in range(2):
        lch = ch0 + sl
        b_gather(lch, sl).wait()
        for jg in range(8):
          parv = lax.bitwise_and(
              idx_raw[pl.ds(pl.multiple_of(lch * 128 + 16 * jg, 16), 16)],
              1) * 64
          rowv = iot + 16 * jg
          for c in range(_NF):
            vals = plsc.load_gather(rows_v.at[sl], [rowv, parv + c])
            b_out[sl, c, pl.ds(16 * jg, 16)] = vals
        b_write(qbase + hoff // 128 + lch, sl).start()

    b_write(qbase + hoff // 128 + nch - 2, 0).wait()
    b_write(qbase + hoff // 128 + nch - 1, 1).wait()


@jax.jit
def _emb(idx_flat, w2):
  mesh = plsc.VectorSubcoreMesh(
      core_axis_name="c", subcore_axis_name="s", num_cores=2, num_subcores=16)
  f = functools.partial(
      pl.kernel,
      mesh=mesh,
      out_type=jax.ShapeDtypeStruct((_NT, _NF, _NB), jnp.float32),
      scratch_types=[
          pltpu.VMEM((_H1 * 128,), jnp.int32),
          pltpu.VMEM((_H1 * 128,), jnp.int32),
          pltpu.VMEM((2, 128, 128), jnp.float32),
          pltpu.VMEM((2, _NF, 128), jnp.float32),
          pltpu.SemaphoreType.DMA,
          pltpu.SemaphoreType.DMA,
          pltpu.SemaphoreType.DMA,
          pltpu.SemaphoreType.DMA,
          pltpu.SemaphoreType.DMA,
      ],
      compiler_params=pltpu.CompilerParams(
          use_tc_tiling_on_sc=True, needs_layout_passes=False),
  )(_body)
  return f(idx_flat, w2)


def kernel(token_ids, weight):
  idx_flat = token_ids.T.astype(jnp.int32).reshape(_NT * _NB)
  w2 = weight.reshape(_V // 2, 128)
  ot = _emb(idx_flat, w2)
  return jnp.transpose(ot, (2, 0, 1))
